# baseline (speedup 1.0000x reference)
import functools

import jax
import jax.numpy as jnp
from jax import lax
from jax.experimental import pallas as pl
from jax.experimental.pallas import tpu as pltpu
from jax.experimental.pallas import tpu_sc as plsc

N_EXAMPLES = 100000
CLASS_NUM = 128
BATCH = 16384

_INFO = plsc.get_sparse_core_info()
_NC = _INFO.num_cores
_NS = _INFO.num_subcores
_LANES = _INFO.num_lanes
_NW = _NC * _NS
_B_PER_W = BATCH // _NW
_CHUNK = 128
_NCHUNK = _B_PER_W // _CHUNK
_UNROLL = 4


def _sc_body(idx_hbm, table_hbm, out_hbm, idx_v, rows_v, sh_v,
             gsem0, gsem1, gsem2, gsem3, csem, dsem):
    sid = lax.axis_index("s")
    wid = sid * _NC + lax.axis_index("c")
    base = wid * _B_PER_W
    shbase = sid * (_B_PER_W // 2)
    pltpu.sync_copy(idx_hbm.at[wid], idx_v)

    gsems = (gsem0, gsem1, gsem2, gsem3)
    gathers = []
    for g in range(_NCHUNK):
        gathers.append(pltpu.async_copy(
            table_hbm.at[idx_v.at[g]],
            rows_v.at[pl.ds(g * _CHUNK, _CHUNK)],
            gsems[g]))

    def sigmoid_rows(lo, n):
        def row(b, carry):
            for j in range(CLASS_NUM // _LANES):
                x = rows_v[lo + b, pl.ds(j * _LANES, _LANES)]
                rows_v[lo + b, pl.ds(j * _LANES, _LANES)] = (
                    1.0 / (1.0 + jnp.exp(-x)))
            return carry

        lax.fori_loop(0, n, row, 0, unroll=_UNROLL)

    cross = {}
    drains = []
    for g in range(_NCHUNK):
        lo = g * _CHUNK
        gathers[g].wait()
        sigmoid_rows(lo, _CHUNK)
        if g % 2 == 0:
            cross[g] = pltpu.async_copy(
                rows_v.at[pl.ds(lo, _CHUNK)],
                sh_v.at[pl.ds(shbase + (g // 2) * _CHUNK, _CHUNK)],
                csem)
        else:
            drains.append(pltpu.async_copy(
                rows_v.at[pl.ds(lo, _CHUNK)],
                out_hbm.at[pl.ds(base + lo, _CHUNK)],
                dsem))
        if g - 1 in cross:
            cross[g - 1].wait()
            plo = (g - 1) * _CHUNK
            drains.append(pltpu.async_copy(
                sh_v.at[pl.ds(shbase + ((g - 1) // 2) * _CHUNK, _CHUNK)],
                out_hbm.at[pl.ds(base + plo, _CHUNK)],
                dsem))
    g = _NCHUNK - 1
    if g in cross:
        cross[g].wait()
        drains.append(pltpu.async_copy(
            sh_v.at[pl.ds(shbase + (g // 2) * _CHUNK, _CHUNK)],
            out_hbm.at[pl.ds(base + g * _CHUNK, _CHUNK)],
            dsem))
    for d in drains:
        d.wait()


@functools.partial(jax.jit)
def kernel(indices, logits):
    mesh = plsc.VectorSubcoreMesh(core_axis_name="c", subcore_axis_name="s")
    run = functools.partial(
        pl.kernel,
        mesh=mesh,
        out_type=jax.ShapeDtypeStruct((BATCH, CLASS_NUM), jnp.float32),
        compiler_params=pltpu.CompilerParams(
            disable_bounds_checks=True, disable_semaphore_checks=True),
        scratch_types=[
            pltpu.VMEM((_NCHUNK, _CHUNK), jnp.int32),
            pltpu.VMEM((_B_PER_W, CLASS_NUM), jnp.float32),
            pltpu.VMEM_SHARED((_NS * (_B_PER_W // 2), CLASS_NUM),
                              jnp.float32),
        ] + [pltpu.SemaphoreType.DMA] * 6,
    )(_sc_body)
    return run(indices.astype(jnp.int32).reshape(_NW, _NCHUNK, _CHUNK), logits)


# speedup vs baseline: 1.0229x; 1.0229x over previous
import functools

import jax
import jax.numpy as jnp
from jax import lax
from jax.experimental import pallas as pl
from jax.experimental.pallas import tpu as pltpu
from jax.experimental.pallas import tpu_sc as plsc

N_EXAMPLES = 100000
CLASS_NUM = 128
BATCH = 16384

_INFO = plsc.get_sparse_core_info()
_NC = _INFO.num_cores
_NS = _INFO.num_subcores
_LANES = _INFO.num_lanes
_NW = _NC * _NS
_B_PER_W = BATCH // _NW
_CHUNK = 128
_NCHUNK = _B_PER_W // _CHUNK
_UNROLL = 4


def _sc_body(idx_hbm, table_hbm, out_hbm, idx_v, rows_v,
             gsem0, gsem1, gsem2, gsem3, ssem):
    wid = lax.axis_index("s") * _NC + lax.axis_index("c")
    base = wid * _B_PER_W
    pltpu.sync_copy(idx_hbm.at[wid], idx_v)

    gsems = (gsem0, gsem1, gsem2, gsem3)
    gathers = []
    for g in range(_NCHUNK):
        gathers.append(pltpu.async_copy(
            table_hbm.at[idx_v.at[g]],
            rows_v.at[pl.ds(g * _CHUNK, _CHUNK)],
            gsems[g]))

    def sigmoid_rows(lo, n):
        def row(b, carry):
            for j in range(CLASS_NUM // _LANES):
                x = rows_v[lo + b, pl.ds(j * _LANES, _LANES)]
                rows_v[lo + b, pl.ds(j * _LANES, _LANES)] = (
                    1.0 / (1.0 + jnp.exp(-x)))
            return carry

        lax.fori_loop(0, n, row, 0, unroll=_UNROLL)

    pieces = [(g * _CHUNK, _CHUNK) for g in range(_NCHUNK)]

    scatters = []
    for lo, n in pieces:
        g = lo // _CHUNK
        if lo % _CHUNK == 0:
            gathers[g].wait()
        sigmoid_rows(lo, n)
        scatters.append(pltpu.async_copy(
            rows_v.at[pl.ds(lo, n)],
            out_hbm.at[pl.ds(base + lo, n)],
            ssem))
    for s in scatters:
        s.wait()


@functools.partial(jax.jit)
def kernel(indices, logits):
    mesh = plsc.VectorSubcoreMesh(core_axis_name="c", subcore_axis_name="s")
    run = functools.partial(
        pl.kernel,
        mesh=mesh,
        out_type=jax.ShapeDtypeStruct((BATCH, CLASS_NUM), jnp.float32),
        scratch_types=[
            pltpu.VMEM((_NCHUNK, _CHUNK), jnp.int32),
            pltpu.VMEM((_B_PER_W, CLASS_NUM), jnp.float32),
        ] + [pltpu.SemaphoreType.DMA] * 5,
    )(_sc_body)
    return run(indices.astype(jnp.int32).reshape(_NW, _NCHUNK, _CHUNK), logits)
